# baseline (speedup 1.0000x reference)
import functools

import jax
import jax.numpy as jnp
from jax import lax
from jax.experimental import pallas as pl
from jax.experimental.pallas import tpu as pltpu
from jax.experimental.pallas import tpu_sc as plsc

GRID = 128
N_PTS = 1048576
V = GRID * GRID * GRID

NC = 2
NS = 16
NW = NC * NS
L = 16

PTS_PER_TILE = N_PTS // NW
P = 1024
CHUNKS = PTS_PER_TILE // P
GROUPS = P // L
NLISTS = 24
NWIN = P // 128
NSTREAMS = NLISTS * NWIN


def _sc_body(xs, ys, zs, chans, out_hbm,
             xv, yv, zv, txv, tyv, tzv, idxv, gathv, outv, sems):
    wid = lax.axis_index("s") * NC + lax.axis_index("c")
    tile_base = wid * PTS_PER_TILE
    iota = lax.iota(jnp.int32, L)

    def compute_and_fire(ci):
        parity = ci & 1
        co = parity * P
        io = parity * NSTREAMS
        go = parity * (NLISTS * P)
        base = tile_base + ci * P
        pltpu.sync_copy(xs.at[pl.ds(base, P)], xv.at[pl.ds(co, P)])
        pltpu.sync_copy(ys.at[pl.ds(base, P)], yv.at[pl.ds(co, P)])
        pltpu.sync_copy(zs.at[pl.ds(base, P)], zv.at[pl.ds(co, P)])

        def idx_body(g, _):
            p0 = g * L
            row = g >> 3
            col = (g & 7) * L

            def axis(ref):
                c = jnp.clip(ref[pl.ds(co + p0, L)] * float(GRID) - 0.5,
                             0.0, float(GRID - 1))
                i0 = c.astype(jnp.int32)
                return i0, c - i0.astype(jnp.float32)

            ix0, tx = axis(xv)
            iy0, ty = axis(yv)
            iz0, tz = axis(zv)
            txv[pl.ds(co + p0, L)] = tx
            tyv[pl.ds(co + p0, L)] = ty
            tzv[pl.ds(co + p0, L)] = tz
            ix1 = jnp.minimum(ix0 + 1, GRID - 1)
            iy1 = (jnp.minimum(iy0 + 1, GRID - 1)) << 7
            iz1 = (jnp.minimum(iz0 + 1, GRID - 1)) << 14
            iy0 = iy0 << 7
            iz0 = iz0 << 14
            corners = (iz0 + iy0 + ix0, iz0 + iy0 + ix1,
                       iz0 + iy1 + ix0, iz0 + iy1 + ix1,
                       iz1 + iy0 + ix0, iz1 + iy0 + ix1,
                       iz1 + iy1 + ix0, iz1 + iy1 + ix1)
            rpos = (p0 + 64) & (P - 1)
            rrow = rpos >> 7
            rcol = rpos & 127
            for c in range(8):
                for ch in range(3):
                    if c & 1:
                        idxv[io + (ch * 8 + c) * 8 + rrow, pl.ds(rcol, L)] = (
                            corners[c] + ch * V)
                    else:
                        idxv[io + (ch * 8 + c) * 8 + row, pl.ds(col, L)] = (
                            corners[c] + ch * V)
            return 0

        lax.fori_loop(0, GROUPS, idx_body, 0)

        def fire(q, _):
            pltpu.async_copy(chans.at[idxv.at[io + q]],
                             gathv.at[pl.ds(go + q * 128, 128)],
                             sems.at[parity])
            return 0

        lax.fori_loop(0, NSTREAMS, fire, 0)

    def finish(ci):
        parity = ci & 1
        co = parity * P
        go = parity * (NLISTS * P)
        base = tile_base + ci * P

        pltpu.make_async_copy(chans.at[pl.ds(0, NLISTS * P)],
                              gathv.at[pl.ds(go, NLISTS * P)],
                              sems.at[parity]).wait()

        def lerp_body(g, _):
            p0 = g * L
            rpos = (p0 + 64) & (P - 1)
            tx = txv[pl.ds(co + p0, L)]
            ty = tyv[pl.ds(co + p0, L)]
            tz = tzv[pl.ds(co + p0, L)]
            for ch in range(3):
                sbase = go + ch * 8 * P

                def cv(c):
                    off = rpos if c & 1 else p0
                    return gathv[pl.ds(sbase + c * P + off, L)]

                def xl(c):
                    v0 = cv(c)
                    return v0 + tx * (cv(c + 1) - v0)

                c0 = xl(0)
                c0 = c0 + ty * (xl(2) - c0)
                c1 = xl(4)
                c1 = c1 + ty * (xl(6) - c1)
                outv[pl.ds(co * 3 + ch * P + p0, L)] = c0 + tz * (c1 - c0)
            return 0

        lax.fori_loop(0, GROUPS, lerp_body, 0)

        for ch in range(3):
            pltpu.sync_copy(outv.at[pl.ds(co * 3 + ch * P, P)],
                            out_hbm.at[pl.ds(ch * N_PTS + base, P)])

    compute_and_fire(0)

    def pipe_body(ci, _):
        compute_and_fire(ci + 1)
        finish(ci)
        return 0

    lax.fori_loop(0, CHUNKS - 1, pipe_body, 0)
    finish(CHUNKS - 1)


@jax.jit
def kernel(pts, tensor):
    chans = tensor.reshape(3 * V)
    xs = pts[:, 0]
    ys = pts[:, 1]
    zs = pts[:, 2]

    mesh = plsc.VectorSubcoreMesh(core_axis_name="c", subcore_axis_name="s")
    out = pl.kernel(
        _sc_body,
        out_type=jax.ShapeDtypeStruct((N_PTS * 3,), jnp.float32),
        mesh=mesh,
        scratch_types=[
            pltpu.VMEM((2 * P,), jnp.float32),
            pltpu.VMEM((2 * P,), jnp.float32),
            pltpu.VMEM((2 * P,), jnp.float32),
            pltpu.VMEM((2 * P,), jnp.float32),
            pltpu.VMEM((2 * P,), jnp.float32),
            pltpu.VMEM((2 * P,), jnp.float32),
            pltpu.VMEM((2 * NSTREAMS, 128), jnp.int32),
            pltpu.VMEM((2 * NLISTS * P,), jnp.float32),
            pltpu.VMEM((2 * 3 * P,), jnp.float32),
            pltpu.SemaphoreType.DMA((2,)),
        ],
    )(xs, ys, zs, chans)
    return out.reshape(3, N_PTS).T


# speedup vs baseline: 1.0024x; 1.0024x over previous
import functools

import jax
import jax.numpy as jnp
from jax import lax
from jax.experimental import pallas as pl
from jax.experimental.pallas import tpu as pltpu
from jax.experimental.pallas import tpu_sc as plsc

GRID = 128
N_PTS = 1048576
V = GRID * GRID * GRID

NC = 2
NS = 16
NW = NC * NS
L = 16

PTS_PER_TILE = N_PTS // NW
P = 1024
CHUNKS = PTS_PER_TILE // P
GROUPS = P // L
NLISTS = 24
NWIN = P // 128
NSTREAMS = NLISTS * NWIN


def _sc_body(xs, ys, zs, chans, out_hbm,
             xv, yv, zv, txv, tyv, tzv, idxv, gathv, outv, sems):
    wid = lax.axis_index("s") * NC + lax.axis_index("c")
    tile_base = wid * PTS_PER_TILE
    iota = lax.iota(jnp.int32, L)

    def compute_and_fire(ci):
        parity = ci & 1
        co = parity * P
        io = parity * NSTREAMS
        go = parity * (NLISTS * P)
        base = tile_base + ci * P
        pltpu.sync_copy(xs.at[pl.ds(base, P)], xv.at[pl.ds(co, P)])
        pltpu.sync_copy(ys.at[pl.ds(base, P)], yv.at[pl.ds(co, P)])
        pltpu.sync_copy(zs.at[pl.ds(base, P)], zv.at[pl.ds(co, P)])

        def idx_body(g, _):
            p0 = g * L
            row = g >> 3
            col = (g & 7) * L

            def axis(ref):
                c = jnp.clip(ref[pl.ds(co + p0, L)] * float(GRID) - 0.5,
                             0.0, float(GRID - 1))
                i0 = c.astype(jnp.int32)
                return i0, c - i0.astype(jnp.float32)

            ix0, tx = axis(xv)
            iy0, ty = axis(yv)
            iz0, tz = axis(zv)
            txv[pl.ds(co + p0, L)] = tx
            tyv[pl.ds(co + p0, L)] = ty
            tzv[pl.ds(co + p0, L)] = tz
            ix1 = jnp.minimum(ix0 + 1, GRID - 1)
            iy1 = (jnp.minimum(iy0 + 1, GRID - 1)) << 7
            iz1 = (jnp.minimum(iz0 + 1, GRID - 1)) << 14
            iy0 = iy0 << 7
            iz0 = iz0 << 14
            corners = (iz0 + iy0 + ix0, iz0 + iy0 + ix1,
                       iz0 + iy1 + ix0, iz0 + iy1 + ix1,
                       iz1 + iy0 + ix0, iz1 + iy0 + ix1,
                       iz1 + iy1 + ix0, iz1 + iy1 + ix1)
            for c in range(8):
                for ch in range(3):
                    idxv[io + (ch * 8 + c) * 8 + row, pl.ds(col, L)] = (
                        corners[c] + ch * V)
            return 0

        lax.fori_loop(0, GROUPS, idx_body, 0)

        def fire(q, _):
            pltpu.async_copy(chans.at[idxv.at[io + q]],
                             gathv.at[pl.ds(go + q * 128, 128)],
                             sems.at[parity])
            return 0

        lax.fori_loop(0, NSTREAMS, fire, 0)

    def finish(ci):
        parity = ci & 1
        co = parity * P
        go = parity * (NLISTS * P)
        base = tile_base + ci * P

        pltpu.make_async_copy(chans.at[pl.ds(0, NLISTS * P)],
                              gathv.at[pl.ds(go, NLISTS * P)],
                              sems.at[parity]).wait()

        def lerp_body(g, _):
            p0 = g * L
            tx = txv[pl.ds(co + p0, L)]
            ty = tyv[pl.ds(co + p0, L)]
            tz = tzv[pl.ds(co + p0, L)]
            for ch in range(3):
                sbase = go + ch * 8 * P + p0

                def cv(c):
                    return gathv[pl.ds(sbase + c * P, L)]

                def xl(c):
                    v0 = cv(c)
                    return v0 + tx * (cv(c + 1) - v0)

                c0 = xl(0)
                c0 = c0 + ty * (xl(2) - c0)
                c1 = xl(4)
                c1 = c1 + ty * (xl(6) - c1)
                outv[pl.ds(co * 3 + ch * P + p0, L)] = c0 + tz * (c1 - c0)
            return 0

        lax.fori_loop(0, GROUPS, lerp_body, 0)

        for ch in range(3):
            pltpu.sync_copy(outv.at[pl.ds(co * 3 + ch * P, P)],
                            out_hbm.at[pl.ds(ch * N_PTS + base, P)])

    compute_and_fire(0)

    def pipe_body(ci, _):
        compute_and_fire(ci + 1)
        finish(ci)
        return 0

    lax.fori_loop(0, CHUNKS - 1, pipe_body, 0)
    finish(CHUNKS - 1)


@jax.jit
def kernel(pts, tensor):
    chans = tensor.reshape(3 * V)
    xs = pts[:, 0]
    ys = pts[:, 1]
    zs = pts[:, 2]

    mesh = plsc.VectorSubcoreMesh(core_axis_name="c", subcore_axis_name="s")
    out = pl.kernel(
        _sc_body,
        out_type=jax.ShapeDtypeStruct((N_PTS * 3,), jnp.float32),
        mesh=mesh,
        scratch_types=[
            pltpu.VMEM((2 * P,), jnp.float32),
            pltpu.VMEM((2 * P,), jnp.float32),
            pltpu.VMEM((2 * P,), jnp.float32),
            pltpu.VMEM((2 * P,), jnp.float32),
            pltpu.VMEM((2 * P,), jnp.float32),
            pltpu.VMEM((2 * P,), jnp.float32),
            pltpu.VMEM((2 * NSTREAMS, 128), jnp.int32),
            pltpu.VMEM((2 * NLISTS * P,), jnp.float32),
            pltpu.VMEM((2 * 3 * P,), jnp.float32),
            pltpu.SemaphoreType.DMA((2,)),
        ],
    )(xs, ys, zs, chans)
    return out.reshape(3, N_PTS).T
